# baseline (speedup 1.0000x reference)
import functools

import jax
import jax.numpy as jnp
from jax import lax
from jax.experimental import pallas as pl
from jax.experimental.pallas import tpu as pltpu
from jax.experimental.pallas import tpu_sc as plsc

_V = 100
_E = 128
_P = 6
_R = 64
_VP = 128
_NW = 32
_BT = 8


def _fused(xs, embed, Wa_w, ua, path_map):
    B, L = xs.shape
    rows_pw = B // _NW
    G = 4
    ngr = rows_pw // G
    tok_pg = G * L
    nblk = -(-tok_pg // 16)
    mesh = plsc.VectorSubcoreMesh(core_axis_name="c", subcore_axis_name="s")

    @functools.partial(
        pl.kernel, mesh=mesh,
        out_type=jax.ShapeDtypeStruct((B, L, _E), jnp.float32),
        compiler_params=pltpu.CompilerParams(needs_layout_passes=False),
        scratch_types=[
            pltpu.VMEM((rows_pw, L), jnp.int32),
            pltpu.VMEM((_VP * _E,), jnp.float32),
            pltpu.VMEM((_V, _E), jnp.float32),
            pltpu.VMEM((_R, 2 * _E), jnp.float32),
            pltpu.VMEM((1, 1, 1, _R), jnp.float32),
            pltpu.VMEM((2 * _E,), jnp.float32),
            pltpu.VMEM((16 * _E,), jnp.float32),
            pltpu.VMEM((_V, _P), jnp.int32),
            pltpu.VMEM((G * L, _E), jnp.float32),
            pltpu.VMEM((G * L, _E), jnp.float32),
            pltpu.VMEM_SHARED((_BT, 16 * _E), jnp.float32),
            pltpu.SemaphoreType.DMA,
            pltpu.SemaphoreType.DMA,
        ],
    )
    def k(emb_hbm, waw_hbm, ua_hbm, pm_hbm, idx_hbm, out_hbm,
          idx_v, tbl_v, emb_v, waw_v, ua_v, w12_v, gpart_v, pm_v, buf0, buf1,
          shared, wsem0, wsem1):
        cid = lax.axis_index("c")
        sid = lax.axis_index("s")
        wid = sid * 2 + cid
        base = wid * rows_pw
        lanes = lax.broadcasted_iota(jnp.int32, (16,), 0)
        zeros16 = jnp.zeros((16,), jnp.float32)

        pltpu.sync_copy(idx_hbm.at[pl.ds(base, rows_pw)], idx_v)
        pltpu.sync_copy(emb_hbm, emb_v)
        pltpu.sync_copy(pm_hbm, pm_v)
        pltpu.sync_copy(ua_hbm, ua_v)
        pltpu.sync_copy(waw_hbm, waw_v)

        def full16(x):
            return jnp.full((16,), x, jnp.int32)

        z16 = full16(0)

        @pl.when(sid < _BT)
        def _build():
            for cb in range(2 * _E // 16):
                def wbody(r, acc, cb=cb):
                    uar = plsc.load_gather(ua_v, [z16, z16, z16, full16(0) + r])
                    wrow = plsc.load_gather(waw_v, [full16(0) + r,
                                                    cb * 16 + lanes])
                    return acc + uar * wrow
                w12_v[pl.ds(cb * 16, 16)] = lax.fori_loop(0, _R, wbody, zeros16)

            vv = sid * 16 + lanes
            vvc = jnp.minimum(vv, _V - 1)
            pj = [plsc.load_gather(pm_v, [vvc, full16(p)]) for p in range(_P)]

            def sbody(c, carry):
                cc = full16(0) + c
                w1c = plsc.load_gather(w12_v, [cc])
                w2c = plsc.load_gather(w12_v, [cc + _E])
                ei = plsc.load_gather(emb_v, [vvc, cc])
                out = []
                for p in range(_P):
                    ej = plsc.load_gather(emb_v, [pj[p], cc])
                    s = carry[2 * p] + jnp.where(ej != 0.0, ei, 0.0) * w1c \
                        + ej * w2c
                    ss = carry[2 * p + 1] + ej
                    out += [s, ss]
                return tuple(out)

            carry = lax.fori_loop(0, _E, sbody, (zeros16,) * (2 * _P))
            neg = jnp.full((16,), -1e10, jnp.float32)
            scores = [jnp.where(carry[2 * p + 1] == 0.0, neg, carry[2 * p])
                      for p in range(_P)]
            m = scores[0]
            for s in scores[1:]:
                m = jnp.maximum(m, s)
            es = [jnp.exp(s - m) for s in scores]
            z = es[0]
            for e in es[1:]:
                z = z + e
            alpha = [e / z for e in es]

            lanebase = lanes * _E

            def gbody(c, _):
                cc = full16(0) + c
                g = zeros16
                for p in range(_P):
                    g = g + alpha[p] * plsc.load_gather(emb_v, [pj[p], cc])
                plsc.store_scatter(gpart_v, [lanebase + c], g)
                return 0

            lax.fori_loop(0, _E, gbody, 0)
            pltpu.sync_copy(gpart_v, shared.at[sid])

        plsc.subcore_barrier()
        for t in range(_BT):
            pltpu.sync_copy(shared.at[t],
                            tbl_v.at[pl.ds(t * 16 * _E, 16 * _E)])

        bufs = (buf0, buf1)
        wsems = (wsem0, wsem1)

        def write_start(g, b):
            for q in range(G):
                pltpu.async_copy(bufs[b].at[pl.ds(q * L, L)],
                                 out_hbm.at[base + g * G + q], wsems[b])

        def write_wait(g, b):
            for q in range(G):
                pltpu.make_async_copy(bufs[b].at[pl.ds(q * L, L)],
                                      out_hbm.at[base + g * G + q],
                                      wsems[b]).wait()

        for g in range(ngr):
            b = g % 2
            if g >= 2:
                write_wait(g - 2, b)

            def body(i, _, g=g, b=b):
                tok = i * 16 + lanes
                msk = tok < tok_pg
                tokc = jnp.minimum(tok, tok_pg - 1)
                gtok = g * tok_pg + tokc
                d0 = gtok // L
                d1 = gtok - d0 * L
                rowv = plsc.load_gather(idx_v, [d0, d1])
                srcbase = rowv * _E
                for c in range(_E):
                    val = plsc.load_gather(tbl_v, [srcbase + c])
                    plsc.store_scatter(bufs[b], [tokc, full16(c)], val,
                                       mask=msk)
                return 0

            lax.fori_loop(0, nblk, body, 0)
            write_start(g, b)
        write_wait(ngr - 2, ngr % 2)
        write_wait(ngr - 1, (ngr - 1) % 2)

    return k(embed, Wa_w, ua, path_map, xs)


def kernel(xs, embed, Wa_w, Wa_b, ua, path_map):
    del Wa_b
    return _fused(xs, embed, Wa_w, ua, path_map)


# speedup vs baseline: 2.2135x; 2.2135x over previous
import functools

import jax
import jax.numpy as jnp
from jax import lax
from jax.experimental import pallas as pl
from jax.experimental.pallas import tpu as pltpu
from jax.experimental.pallas import tpu_sc as plsc

_V = 100
_E = 128
_ES = _E + 1
_SLAB = 2176
_SPAD = _SLAB - 16 * _ES
_P = 6
_R = 64
_VP = 128
_NW = 32
_BT = 8


def _fused(xs, embed, Wa_w, ua, path_map):
    B, L = xs.shape
    rows_pw = B // _NW
    mesh = plsc.VectorSubcoreMesh(core_axis_name="c", subcore_axis_name="s")

    @functools.partial(
        pl.kernel, mesh=mesh,
        out_type=jax.ShapeDtypeStruct((B, L, _E), jnp.float32),
        compiler_params=pltpu.CompilerParams(needs_layout_passes=False),
        scratch_types=[
            pltpu.VMEM((rows_pw, L), jnp.int32),
            pltpu.VMEM((_BT * _SLAB,), jnp.float32),
            pltpu.VMEM((_V, _E), jnp.float32),
            pltpu.VMEM((_V * _ES,), jnp.float32),
            pltpu.VMEM((_R, 2 * _E), jnp.float32),
            pltpu.VMEM((1, 1, 1, _R), jnp.float32),
            pltpu.VMEM((2 * _E,), jnp.float32),
            pltpu.VMEM((_SLAB,), jnp.float32),
            pltpu.VMEM((_V, _P), jnp.int32),
            pltpu.VMEM((L, _E), jnp.float32),
            pltpu.VMEM((L, _E), jnp.float32),
            pltpu.VMEM_SHARED((_BT, _SLAB), jnp.float32),
            pltpu.SemaphoreType.DMA,
            pltpu.SemaphoreType.DMA,
        ],
    )
    def k(emb_hbm, waw_hbm, ua_hbm, pm_hbm, idx_hbm, out_hbm,
          idx_v, tbl_v, emb_v, emb2_v, waw_v, ua_v, w12_v, gpart_v,
          pm_v, buf0, buf1, shared, wsem0, wsem1):
        cid = lax.axis_index("c")
        sid = lax.axis_index("s")
        wid = sid * 2 + cid
        base = wid * rows_pw
        lanes = lax.broadcasted_iota(jnp.int32, (16,), 0)
        zeros16 = jnp.zeros((16,), jnp.float32)

        pltpu.sync_copy(idx_hbm.at[pl.ds(base, rows_pw)], idx_v)
        pltpu.sync_copy(emb_hbm, emb_v)
        pltpu.sync_copy(pm_hbm, pm_v)
        pltpu.sync_copy(ua_hbm, ua_v)
        pltpu.sync_copy(waw_hbm, waw_v)

        def full16(x):
            return jnp.full((16,), x, jnp.int32)

        z16 = full16(0)

        @pl.when(sid < _BT)
        def _build():
            def skbody(r, _):
                rb = r * _ES
                for cb in range(_E // 16):
                    v = emb_v[r, pl.ds(cb * 16, 16)]
                    plsc.store_scatter(emb2_v, [rb + (cb * 16) + lanes], v)
                return 0
            lax.fori_loop(0, _V, skbody, 0)

            for cb in range(2 * _E // 16):
                def wbody(r, acc, cb=cb):
                    uar = plsc.load_gather(ua_v, [z16, z16, z16, z16 + r])
                    wrow = plsc.load_gather(waw_v, [z16 + r, cb * 16 + lanes])
                    return acc + uar * wrow
                w12_v[pl.ds(cb * 16, 16)] = lax.fori_loop(0, _R, wbody, zeros16)

            vv = sid * 16 + lanes
            vvc = jnp.minimum(vv, _V - 1)
            vvs = vvc * _ES
            pj = [plsc.load_gather(pm_v, [vvc, full16(p)]) for p in range(_P)]
            pjs = [p_ * _ES for p_ in pj]

            def sbody(c, carry):
                cc = z16 + c
                w1c = plsc.load_gather(w12_v, [cc])
                w2c = plsc.load_gather(w12_v, [cc + _E])
                ei = plsc.load_gather(emb2_v, [vvs + c])
                out = []
                for p in range(_P):
                    ej = plsc.load_gather(emb2_v, [pjs[p] + c])
                    s = carry[2 * p] + jnp.where(ej != 0.0, ei, 0.0) * w1c \
                        + ej * w2c
                    ss = carry[2 * p + 1] + ej
                    out += [s, ss]
                return tuple(out)

            carry = lax.fori_loop(0, _E, sbody, (zeros16,) * (2 * _P))
            neg = jnp.full((16,), -1e10, jnp.float32)
            scores = [jnp.where(carry[2 * p + 1] == 0.0, neg, carry[2 * p])
                      for p in range(_P)]
            m = scores[0]
            for s in scores[1:]:
                m = jnp.maximum(m, s)
            es = [jnp.exp(s - m) for s in scores]
            z = es[0]
            for e in es[1:]:
                z = z + e
            alpha = [e / z for e in es]

            lanebase = lanes * _ES

            def gbody(c, _):
                g = zeros16
                for p in range(_P):
                    g = g + alpha[p] * plsc.load_gather(emb2_v, [pjs[p] + c])
                plsc.store_scatter(gpart_v, [lanebase + c], g)
                return 0

            lax.fori_loop(0, _E, gbody, 0)
            pltpu.sync_copy(gpart_v, shared.at[sid])

        plsc.subcore_barrier()
        for t in range(_BT):
            pltpu.sync_copy(shared.at[t],
                            tbl_v.at[pl.ds(t * _SLAB, _SLAB)])

        bufs = (buf0, buf1)
        wsems = (wsem0, wsem1)
        cvecs = [(cb * 16) + lanes for cb in range(_E // 16)]

        def fill_row(r, b):
            nfull = L // 16
            rem = L - nfull * 16
            for k in range(nfull + (1 if rem else 0)):
                off = k * 16 if k < nfull else L - 16
                rowv = idx_v[r, pl.ds(off, 16)]
                for j in range(16):
                    if k == nfull and j < 16 - rem:
                        continue
                    t = off + j
                    rid = rowv[j]
                    srcv = z16 + (rid * _ES + (rid // 16) * _SPAD)
                    for cb in range(_E // 16):
                        val = plsc.load_gather(tbl_v, [srcv + cvecs[cb]])
                        bufs[b][t, pl.ds(cb * 16, 16)] = val

        def write_start(r, b):
            pltpu.async_copy(bufs[b], out_hbm.at[base + r], wsems[b])

        def write_wait(r, b):
            pltpu.make_async_copy(bufs[b], out_hbm.at[base + r],
                                  wsems[b]).wait()

        def pbody(i, _):
            r0 = 2 * i

            @pl.when(i >= 1)
            def _():
                write_wait(r0 - 2, 0)
            fill_row(r0, 0)
            write_start(r0, 0)

            @pl.when(i >= 1)
            def _():
                write_wait(r0 - 1, 1)
            fill_row(r0 + 1, 1)
            write_start(r0 + 1, 1)
            return 0

        lax.fori_loop(0, rows_pw // 2, pbody, 0)
        write_wait(rows_pw - 2, 0)
        write_wait(rows_pw - 1, 1)

    return k(embed, Wa_w, ua, path_map, xs)


def kernel(xs, embed, Wa_w, Wa_b, ua, path_map):
    del Wa_b
    return _fused(xs, embed, Wa_w, ua, path_map)


# speedup vs baseline: 3.0201x; 1.3644x over previous
import functools

import jax
import jax.numpy as jnp
from jax import lax
from jax.experimental import pallas as pl
from jax.experimental.pallas import tpu as pltpu
from jax.experimental.pallas import tpu_sc as plsc

_V = 100
_E = 128
_P = 6
_R = 64
_VP = 128
_NW = 32
_CHUNK = 128


def _table_body(embed_ref, pjb_ref, uab_ref, waw_ref, g_ref):
    emb = embed_ref[...]
    w12 = jnp.sum(uab_ref[...] * waw_ref[...], axis=0, keepdims=True)
    w1 = w12[:, :_E]
    w2 = w12[:, _E:]
    iota_u = lax.broadcasted_iota(jnp.int32, (_VP, _VP), 1)
    scores = []
    ejs_all = []
    for p in range(_P):
        pj = pjb_ref[p * _VP:(p + 1) * _VP, :]
        oh = (pj == iota_u).astype(jnp.float32)
        ejs = jnp.dot(oh, emb, preferred_element_type=jnp.float32)
        ei_m = jnp.where(ejs != 0.0, emb, 0.0)
        s = jnp.sum(ei_m * w1 + ejs * w2, axis=1, keepdims=True)
        amask = jnp.sum(ejs, axis=1, keepdims=True) == 0.0
        scores.append(jnp.where(amask, -1e10, s))
        ejs_all.append(ejs)
    m = scores[0]
    for s in scores[1:]:
        m = jnp.maximum(m, s)
    es = [jnp.exp(s - m) for s in scores]
    z = es[0]
    for e in es[1:]:
        z = z + e
    g = (es[0] / z) * ejs_all[0]
    for e, ejs in zip(es[1:], ejs_all[1:]):
        g = g + (e / z) * ejs
    g_ref[...] = g


def _build_table(embed, Wa_w, ua, path_map):
    embed_p = jnp.pad(embed, ((0, _VP - _V), (0, 0)))
    pm_p = jnp.pad(path_map, ((0, _VP - _V), (0, 0)))
    pjb = jnp.broadcast_to(pm_p.T.reshape(_P * _VP, 1), (_P * _VP, _VP))
    uab = jnp.broadcast_to(ua.reshape(_R, 1), (_R, 2 * _E))
    return pl.pallas_call(
        _table_body,
        out_shape=jax.ShapeDtypeStruct((_VP, _E), jnp.float32),
    )(embed_p, pjb, uab, Wa_w)


def _gather_rows(table, xs):
    B, L = xs.shape
    rows_pw = B // _NW
    mesh = plsc.VectorSubcoreMesh(core_axis_name="c", subcore_axis_name="s")

    G = 8
    ngr = rows_pw // G

    @functools.partial(
        pl.kernel, mesh=mesh,
        out_type=jax.ShapeDtypeStruct((B, L, _E), jnp.float32),
        compiler_params=pltpu.CompilerParams(use_tc_tiling_on_sc=True,
                                             needs_layout_passes=True),
        scratch_types=[
            pltpu.VMEM((rows_pw, L), jnp.int32),
            pltpu.VMEM((G, L, _E), jnp.float32),
            pltpu.VMEM((G, L, _E), jnp.float32),
            pltpu.SemaphoreType.DMA,
            pltpu.SemaphoreType.DMA,
            pltpu.SemaphoreType.DMA,
            pltpu.SemaphoreType.DMA,
        ],
    )
    def k(table_hbm, idx_hbm, out_hbm, idx_v, buf0, buf1,
          gsem0, gsem1, wsem0, wsem1):
        wid = lax.axis_index("s") * 2 + lax.axis_index("c")
        base = wid * rows_pw
        pltpu.sync_copy(idx_hbm.at[pl.ds(base, rows_pw)], idx_v)

        bufs = (buf0, buf1)
        gsems = (gsem0, gsem1)
        wsems = (wsem0, wsem1)

        def fire(g, b):
            for q in range(G):
                pltpu.async_copy(table_hbm.at[idx_v.at[g * G + q]],
                                 bufs[b].at[q], gsems[b])

        def drain(g, b):
            for q in range(G):
                pltpu.make_async_copy(table_hbm.at[idx_v.at[g * G + q]],
                                      bufs[b].at[q], gsems[b]).wait()

        def write_start(g, b):
            pltpu.async_copy(bufs[b], out_hbm.at[pl.ds(base + g * G, G)],
                             wsems[b])

        def write_wait(g, b):
            pltpu.make_async_copy(bufs[b], out_hbm.at[pl.ds(base + g * G, G)],
                                  wsems[b]).wait()

        fire(0, 0)
        for g in range(ngr):
            b = g % 2
            if g + 1 < ngr:
                if g >= 1:
                    write_wait(g - 1, 1 - b)
                fire(g + 1, 1 - b)
            drain(g, b)
            write_start(g, b)
        write_wait(ngr - 1, (ngr - 1) % 2)
        if ngr >= 2:
            write_wait(ngr - 2, ngr % 2)

    return k(table, xs)


def kernel(xs, embed, Wa_w, Wa_b, ua, path_map):
    del Wa_b
    B, L = xs.shape
    table = _build_table(embed, Wa_w, ua, path_map)
    assert B % _NW == 0
    return _gather_rows(table, xs)
